# baseline (speedup 1.0000x reference)
import functools

import jax
import jax.numpy as jnp
from jax.experimental import pallas as pl
from jax.experimental.pallas import tpu as pltpu
from jax.experimental.pallas import tpu_sc as plsc

_B = 16384
_D = 128
_GW = 128
_BS = 4096


def _sc_gather(user_idx, item_idx, user_table, item_table, lo, n):
    nw = 32
    per = n // nw
    nch = per // _GW
    mesh = plsc.VectorSubcoreMesh(core_axis_name="core",
                                  subcore_axis_name="subcore")

    @functools.partial(
        pl.kernel,
        out_type=(jax.ShapeDtypeStruct((n, _D), jnp.float32),
                  jax.ShapeDtypeStruct((n, _D), jnp.float32)),
        mesh=mesh,
        scratch_types=(
            [pltpu.VMEM((per,), jnp.int32) for _ in range(2)]
            + [pltpu.VMEM((per, _D), jnp.float32) for _ in range(2)]
            + [pltpu.SemaphoreType.DMA for _ in range(2 * nch)]),
    )
    def gather_kernel(ut_hbm, it_hbm, ui_hbm, ii_hbm, eu_hbm, ev_hbm, *scr):
        idx_bufs = scr[:2]
        row_bufs = scr[2:4]
        sems = scr[4:]
        wid = jax.lax.axis_index("subcore") * 2 + jax.lax.axis_index("core")
        src_base = lo + wid * per
        dst_base = wid * per

        copies = []
        for t, idx_hbm, tab in ((0, ui_hbm, ut_hbm), (1, ii_hbm, it_hbm)):
            pltpu.sync_copy(idx_hbm.at[pl.ds(src_base, per)], idx_bufs[t])
            for c in range(nch):
                copies.append(
                    pltpu.async_copy(
                        tab.at[idx_bufs[t].at[pl.ds(c * _GW, _GW)]],
                        row_bufs[t].at[pl.ds(c * _GW, _GW)],
                        sems[t * nch + c]))
        for t, out_hbm in ((0, eu_hbm), (1, ev_hbm)):
            for c in range(nch):
                copies[t * nch + c].wait()
            pltpu.sync_copy(row_bufs[t], out_hbm.at[pl.ds(dst_base, per)])

    return gather_kernel(user_table, item_table, user_idx, item_idx)


def _mlp_body(eu_ref, ev_ref, w1_ref, b1_ref, w2_ref, b2_ref,
              w3_ref, b3_ref, w4_ref, b4_ref, wp_ref, bp_ref, out_ref):
    def dot(a, w):
        return jnp.dot(a, w, preferred_element_type=jnp.float32)

    x = dot(eu_ref[...], w1_ref[0:_D, :]) + dot(ev_ref[...], w1_ref[_D:, :])
    x = jnp.maximum(x + b1_ref[...], 0.0)
    x = jnp.maximum(dot(x, w2_ref[...]) + b2_ref[...], 0.0)
    x = jnp.maximum(dot(x, w3_ref[...]) + b3_ref[...], 0.0)
    x = jnp.maximum(dot(x, w4_ref[...]) + b4_ref[...], 0.0)
    out_ref[...] = (dot(x, wp_ref[...]) + bp_ref[...]).reshape(-1)


def _mlp(eu, ev, w1, b1, w2, b2, w3, b3, w4, b4, wp, bp):
    n = eu.shape[0]
    bs = min(_BS, n)

    def _full(a):
        return pl.BlockSpec(a.shape, lambda i: (0,) * a.ndim)

    return pl.pallas_call(
        _mlp_body,
        grid=(n // bs,),
        in_specs=[
            pl.BlockSpec((bs, _D), lambda i: (i, 0)),
            pl.BlockSpec((bs, _D), lambda i: (i, 0)),
            _full(w1), _full(b1), _full(w2), _full(b2),
            _full(w3), _full(b3), _full(w4), _full(b4), _full(wp), _full(bp),
        ],
        out_specs=pl.BlockSpec((bs,), lambda i: (i,)),
        out_shape=jax.ShapeDtypeStruct((n,), jnp.float32),
        compiler_params=pltpu.CompilerParams(
            dimension_semantics=("arbitrary",)),
    )(eu, ev, w1, b1, w2, b2, w3, b3, w4, b4, wp, bp)


def kernel(user, item, user_table, item_table,
           W1, b1, W2, b2, W3, b3, W4, b4, Wp, bp):
    user = user.astype(jnp.int32)
    item = item.astype(jnp.int32)
    h = _B // 2
    outs = []
    for lo in (0, h):
        eu, ev = _sc_gather(user, item, user_table, item_table, lo, h)
        outs.append(_mlp(eu, ev, W1, b1, W2, b2, W3, b3, W4, b4, Wp, bp))
    return jnp.concatenate(outs)


# speedup vs baseline: 1.0070x; 1.0070x over previous
import functools

import jax
import jax.numpy as jnp
from jax.experimental import pallas as pl
from jax.experimental.pallas import tpu as pltpu
from jax.experimental.pallas import tpu_sc as plsc

_B = 16384
_D = 128
_GW = 128
_BS = 4096


def _sc_gather(user_idx, item_idx, user_table, item_table, lo, n):
    nw = 32
    per = n // nw
    nch = per // _GW
    mesh = plsc.VectorSubcoreMesh(core_axis_name="core",
                                  subcore_axis_name="subcore")

    @functools.partial(
        pl.kernel,
        out_type=(jax.ShapeDtypeStruct((n, _D), jnp.float32),
                  jax.ShapeDtypeStruct((n, _D), jnp.float32)),
        mesh=mesh,
        scratch_types=(
            [pltpu.VMEM((per,), jnp.int32) for _ in range(2)]
            + [pltpu.VMEM((per, _D), jnp.float32) for _ in range(2)]
            + [pltpu.SemaphoreType.DMA for _ in range(2 * nch)]),
    )
    def gather_kernel(ut_hbm, it_hbm, ui_hbm, ii_hbm, eu_hbm, ev_hbm, *scr):
        idx_bufs = scr[:2]
        row_bufs = scr[2:4]
        sems = scr[4:]
        wid = jax.lax.axis_index("subcore") * 2 + jax.lax.axis_index("core")
        src_base = lo + wid * per
        dst_base = wid * per

        copies = []
        for t, idx_hbm, tab in ((0, ui_hbm, ut_hbm), (1, ii_hbm, it_hbm)):
            pltpu.sync_copy(idx_hbm.at[pl.ds(src_base, per)], idx_bufs[t])
            for c in range(nch):
                copies.append(
                    pltpu.async_copy(
                        tab.at[idx_bufs[t].at[pl.ds(c * _GW, _GW)]],
                        row_bufs[t].at[pl.ds(c * _GW, _GW)],
                        sems[t * nch + c]))
        for t, out_hbm in ((0, eu_hbm), (1, ev_hbm)):
            for c in range(nch):
                copies[t * nch + c].wait()
                pltpu.sync_copy(
                    row_bufs[t].at[pl.ds(c * _GW, _GW)],
                    out_hbm.at[pl.ds(dst_base + c * _GW, _GW)])

    return gather_kernel(user_table, item_table, user_idx, item_idx)


def _mlp_body(eu_ref, ev_ref, w1_ref, b1_ref, w2_ref, b2_ref,
              w3_ref, b3_ref, w4_ref, b4_ref, wp_ref, bp_ref, out_ref):
    def dot(a, w):
        return jnp.dot(a, w, preferred_element_type=jnp.float32)

    x = dot(eu_ref[...], w1_ref[0:_D, :]) + dot(ev_ref[...], w1_ref[_D:, :])
    x = jnp.maximum(x + b1_ref[...], 0.0)
    x = jnp.maximum(dot(x, w2_ref[...]) + b2_ref[...], 0.0)
    x = jnp.maximum(dot(x, w3_ref[...]) + b3_ref[...], 0.0)
    x = jnp.maximum(dot(x, w4_ref[...]) + b4_ref[...], 0.0)
    out_ref[...] = (dot(x, wp_ref[...]) + bp_ref[...]).reshape(-1)


def _mlp(eu, ev, w1, b1, w2, b2, w3, b3, w4, b4, wp, bp):
    n = eu.shape[0]
    bs = min(_BS, n)

    def _full(a):
        return pl.BlockSpec(a.shape, lambda i: (0,) * a.ndim)

    return pl.pallas_call(
        _mlp_body,
        grid=(n // bs,),
        in_specs=[
            pl.BlockSpec((bs, _D), lambda i: (i, 0)),
            pl.BlockSpec((bs, _D), lambda i: (i, 0)),
            _full(w1), _full(b1), _full(w2), _full(b2),
            _full(w3), _full(b3), _full(w4), _full(b4), _full(wp), _full(bp),
        ],
        out_specs=pl.BlockSpec((bs,), lambda i: (i,)),
        out_shape=jax.ShapeDtypeStruct((n,), jnp.float32),
        compiler_params=pltpu.CompilerParams(
            dimension_semantics=("arbitrary",)),
    )(eu, ev, w1, b1, w2, b2, w3, b3, w4, b4, wp, bp)


def kernel(user, item, user_table, item_table,
           W1, b1, W2, b2, W3, b3, W4, b4, Wp, bp):
    user = user.astype(jnp.int32)
    item = item.astype(jnp.int32)
    h = _B // 2
    outs = []
    for lo in (0, h):
        eu, ev = _sc_gather(user, item, user_table, item_table, lo, h)
        outs.append(_mlp(eu, ev, W1, b1, W2, b2, W3, b3, W4, b4, Wp, bp))
    return jnp.concatenate(outs)


# speedup vs baseline: 1.0382x; 1.0310x over previous
import functools

import jax
import jax.numpy as jnp
from jax.experimental import pallas as pl
from jax.experimental.pallas import tpu as pltpu
from jax.experimental.pallas import tpu_sc as plsc

_B = 16384
_D = 128
_GW = 128
_BS = 4096


def _sc_gather(user_idx, item_idx, user_table, item_table, lo, n):
    nw = 32
    per = n // nw
    nch = per // _GW
    mesh = plsc.VectorSubcoreMesh(core_axis_name="core",
                                  subcore_axis_name="subcore")

    @functools.partial(
        pl.kernel,
        out_type=(jax.ShapeDtypeStruct((n, _D), jnp.float32),
                  jax.ShapeDtypeStruct((n, _D), jnp.float32)),
        mesh=mesh,
        scratch_types=(
            [pltpu.VMEM((per,), jnp.int32) for _ in range(2)]
            + [pltpu.VMEM((per, _D), jnp.float32) for _ in range(2)]
            + [pltpu.SemaphoreType.DMA for _ in range(2 * nch)]),
    )
    def gather_kernel(ut_hbm, it_hbm, ui_hbm, ii_hbm, eu_hbm, ev_hbm, *scr):
        idx_bufs = scr[:2]
        row_bufs = scr[2:4]
        sems = scr[4:]
        wid = jax.lax.axis_index("subcore") * 2 + jax.lax.axis_index("core")
        src_base = lo + wid * per
        dst_base = wid * per

        copies = []
        for t, idx_hbm, tab in ((0, ui_hbm, ut_hbm), (1, ii_hbm, it_hbm)):
            pltpu.sync_copy(idx_hbm.at[pl.ds(src_base, per)], idx_bufs[t])
            for c in range(nch):
                copies.append(
                    pltpu.async_copy(
                        tab.at[idx_bufs[t].at[pl.ds(c * _GW, _GW)]],
                        row_bufs[t].at[pl.ds(c * _GW, _GW)],
                        sems[t * nch + c]))
        for t, out_hbm in ((0, eu_hbm), (1, ev_hbm)):
            for c in range(nch):
                copies[t * nch + c].wait()
                pltpu.sync_copy(
                    row_bufs[t].at[pl.ds(c * _GW, _GW)],
                    out_hbm.at[pl.ds(dst_base + c * _GW, _GW)])

    return gather_kernel(user_table, item_table, user_idx, item_idx)


def _mlp_body(eu_ref, ev_ref, w1_ref, b1_ref, w2_ref, b2_ref,
              w3_ref, b3_ref, w4_ref, b4_ref, wp_ref, bp_ref, out_ref):
    def dot(a, w):
        return jnp.dot(a, w, preferred_element_type=jnp.float32)

    x = dot(eu_ref[...], w1_ref[0:_D, :]) + dot(ev_ref[...], w1_ref[_D:, :])
    x = jnp.maximum(x + b1_ref[...], 0.0)
    x = jnp.maximum(dot(x, w2_ref[...]) + b2_ref[...], 0.0)
    x = jnp.maximum(dot(x, w3_ref[...]) + b3_ref[...], 0.0)
    x = jnp.maximum(dot(x, w4_ref[...]) + b4_ref[...], 0.0)
    out_ref[...] = (dot(x, wp_ref[...]) + bp_ref[...]).reshape(-1)


def _mlp(eu, ev, w1, b1, w2, b2, w3, b3, w4, b4, wp, bp, y_prev, lo):
    n = eu.shape[0]
    bs = min(_BS, n)
    blk_off = lo // bs

    def _full(a):
        return pl.BlockSpec(a.shape, lambda i: (0,) * a.ndim)

    ins = [eu, ev, w1, b1, w2, b2, w3, b3, w4, b4, wp, bp]
    in_specs = [
        pl.BlockSpec((bs, _D), lambda i: (i, 0)),
        pl.BlockSpec((bs, _D), lambda i: (i, 0)),
        _full(w1), _full(b1), _full(w2), _full(b2),
        _full(w3), _full(b3), _full(w4), _full(b4), _full(wp), _full(bp),
    ]
    body = _mlp_body
    aliases = {}
    if y_prev is not None:
        ins.append(y_prev)
        in_specs.append(pl.BlockSpec(memory_space=pl.ANY))
        aliases = {12: 0}
        body = lambda *refs: _mlp_body(*refs[:12], refs[13])

    return pl.pallas_call(
        body,
        grid=(n // bs,),
        in_specs=in_specs,
        out_specs=pl.BlockSpec((bs,), lambda i: (i + blk_off,)),
        out_shape=jax.ShapeDtypeStruct((_B,), jnp.float32),
        input_output_aliases=aliases,
        compiler_params=pltpu.CompilerParams(
            dimension_semantics=("arbitrary",)),
    )(*ins)


def kernel(user, item, user_table, item_table,
           W1, b1, W2, b2, W3, b3, W4, b4, Wp, bp):
    user = user.astype(jnp.int32)
    item = item.astype(jnp.int32)
    h = _B // 2
    y = None
    for lo in (0, h):
        eu, ev = _sc_gather(user, item, user_table, item_table, lo, h)
        y = _mlp(eu, ev, W1, b1, W2, b2, W3, b3, W4, b4, Wp, bp, y, lo)
    return y
